# baseline (speedup 1.0000x reference)
import functools

import numpy as np
import jax
import jax.numpy as jnp
from jax.experimental import pallas as pl
from jax.experimental.pallas import tpu as pltpu
from jax.experimental.pallas import tpu_sc as plsc

_DIM = 64
_SEQ = 200
_BATCH = 4096
_HALF = 100
_LANES = 16


def _positional_encoding(length, dim):
    depth = dim / 2
    positions = np.arange(length)[:, np.newaxis]
    depths = np.arange(int(depth))[np.newaxis, :] / depth
    angle_rates = 1 / 10000 ** depths
    angle_rads = positions * angle_rates
    return np.concatenate(
        [np.sin(angle_rads), np.cos(angle_rads)], axis=-1
    ).astype(np.float32)


_POS = _positional_encoding(_SEQ, _DIM)


@jax.jit
def _embed_sc(x2, table, pos):
    mesh = plsc.VectorSubcoreMesh(
        core_axis_name="core", subcore_axis_name="subcore"
    )

    @functools.partial(
        pl.kernel,
        out_type=jax.ShapeDtypeStruct((_BATCH * _SEQ, _DIM), jnp.float32),
        mesh=mesh,
        scratch_types=[pltpu.VMEM((_SEQ, _DIM), jnp.float32)],
        compiler_params=pltpu.CompilerParams(use_tc_tiling_on_sc=False),
    )
    def k(x_hbm, table_hbm, pos_hbm, o_hbm, pos_vmem):
        pltpu.sync_copy(pos_hbm, pos_vmem)

        def body(i_vmem, o_vmem):
            pltpu.sync_copy(
                table_hbm.at[i_vmem.at[0]], o_vmem.at[pl.ds(0, _HALF)]
            )
            pltpu.sync_copy(
                table_hbm.at[i_vmem.at[1]], o_vmem.at[pl.ds(_HALF, _HALF)]
            )

            @pl.loop(0, _SEQ)
            def _(r):
                row_o = o_vmem.at[r]
                row_p = pos_vmem.at[r]
                for c in range(0, _DIM, _LANES):
                    row_o.at[pl.ds(c, _LANES)][...] = (
                        row_o.at[pl.ds(c, _LANES)][...] * 8.0
                        + row_p.at[pl.ds(c, _LANES)][...]
                    )

        pltpu.emit_pipeline(
            body,
            grid=(_BATCH,),
            in_specs=[pl.BlockSpec((2, _HALF), index_map=lambda s: (s, 0))],
            out_specs=[pl.BlockSpec((_SEQ, _DIM), index_map=lambda s: (s, 0))],
            core_axis_name=("core", "subcore"),
            dimension_semantics=(pltpu.PARALLEL,),
        )(x_hbm, o_hbm)

    return k(x2, table, pos)


def kernel(x, table):
    x2 = x.astype(jnp.int32).reshape(_BATCH * 2, _HALF)
    out = _embed_sc(x2, table, _POS)
    return out.reshape(_BATCH, _SEQ, _DIM)


# speedup vs baseline: 1.5894x; 1.5894x over previous
import functools

import numpy as np
import jax
from jax import lax
import jax.numpy as jnp
from jax.experimental import pallas as pl
from jax.experimental.pallas import tpu as pltpu
from jax.experimental.pallas import tpu_sc as plsc

_DIM = 64
_SEQ = 200
_BATCH = 4096
_LANES = 16
_NW = 32
_SPW = _BATCH // _NW
_G0 = 104
_G1 = _SEQ - _G0


def _positional_encoding(length, dim):
    depth = dim / 2
    positions = np.arange(length)[:, np.newaxis]
    depths = np.arange(int(depth))[np.newaxis, :] / depth
    angle_rates = 1 / 10000 ** depths
    angle_rads = positions * angle_rates
    return np.concatenate(
        [np.sin(angle_rads), np.cos(angle_rads)], axis=-1
    ).astype(np.float32)


_POS = _positional_encoding(_SEQ, _DIM)


@jax.jit
def _embed_sc(x2, table, pos):
    mesh = plsc.VectorSubcoreMesh(
        core_axis_name="core", subcore_axis_name="subcore"
    )

    @functools.partial(
        pl.kernel,
        out_type=jax.ShapeDtypeStruct((_BATCH * _SEQ, _DIM), jnp.float32),
        mesh=mesh,
        scratch_types=[
            pltpu.VMEM((_SEQ, _DIM), jnp.float32),
            pltpu.VMEM((_SPW, _SEQ), jnp.int32),
            [pltpu.VMEM((_SEQ, _DIM), jnp.float32) for _ in range(4)],
            [pltpu.SemaphoreType.DMA for _ in range(4)],
            [pltpu.SemaphoreType.DMA for _ in range(4)],
        ],
        compiler_params=pltpu.CompilerParams(use_tc_tiling_on_sc=False),
    )
    def k(x_hbm, table_hbm, pos_hbm, o_hbm, pos_v, idx_v, rows, gsems, osems):
        cid = lax.axis_index("core")
        sid = lax.axis_index("subcore")
        base = (sid * 2 + cid) * _SPW

        pltpu.sync_copy(pos_hbm, pos_v)
        pltpu.sync_copy(x_hbm.at[pl.ds(base, _SPW)], idx_v)

        def g_copies(q, b):
            idx = idx_v.at[q]
            return (
                pltpu.make_async_copy(
                    table_hbm.at[idx.at[pl.ds(0, _G0)]],
                    rows[b].at[pl.ds(0, _G0)],
                    gsems[b],
                ),
                pltpu.make_async_copy(
                    table_hbm.at[idx.at[pl.ds(_G0, _G1)]],
                    rows[b].at[pl.ds(_G0, _G1)],
                    gsems[b],
                ),
            )

        def start_gather(q, b):
            for c in g_copies(q, b):
                c.start()

        def wait_gather(q, b):
            for c in g_copies(q, b):
                c.wait()

        def out_copy(q, b):
            row0 = (base + q) * _SEQ
            return pltpu.make_async_copy(
                rows[b], o_hbm.at[pl.ds(row0, _SEQ)], osems[b]
            )

        def compute(b):
            dst = rows[b]

            @plsc.parallel_loop(0, _SEQ, step=2, unroll=4)
            def _(r):
                for rr in range(2):
                    row_o = dst.at[r + rr]
                    row_p = pos_v.at[r + rr]
                    for c in range(0, _DIM, _LANES):
                        row_o.at[pl.ds(c, _LANES)][...] = (
                            row_o.at[pl.ds(c, _LANES)][...] * 8.0
                            + row_p.at[pl.ds(c, _LANES)][...]
                        )

        start_gather(0, 0)
        start_gather(1, 1)

        @pl.loop(0, _SPW, step=4)
        def _(j):
            for b in range(4):
                q = j + b
                wait_gather(q, b)
                compute(b)
                out_copy(q, b).start()
                b2 = (b + 2) % 4
                q2 = q + 2

                @pl.when(q2 < _SPW)
                def _():
                    @pl.when(q2 >= 4)
                    def _():
                        out_copy(q2 - 4, b2).wait()

                    start_gather(q2, b2)

        for b in range(4):
            out_copy(_SPW - 4 + b, b).wait()

    return k(x2, table, pos)


def kernel(x, table):
    x2 = x.astype(jnp.int32)
    out = _embed_sc(x2, table, _POS)
    return out.reshape(_BATCH, _SEQ, _DIM)


# speedup vs baseline: 1.9419x; 1.2218x over previous
import functools

import numpy as np
import jax
from jax import lax
import jax.numpy as jnp
from jax.experimental import pallas as pl
from jax.experimental.pallas import tpu as pltpu
from jax.experimental.pallas import tpu_sc as plsc

_DIM = 64
_PAD = 128
_SEQ = 200
_BATCH = 4096
_LANES = 16
_NW = 32
_SPW = _BATCH // _NW
_G0 = 104
_G1 = _SEQ - _G0
_ITEMS = _SPW * 2
_RING = 4


def _positional_encoding(length, dim):
    depth = dim / 2
    positions = np.arange(length)[:, np.newaxis]
    depths = np.arange(int(depth))[np.newaxis, :] / depth
    angle_rates = 1 / 10000 ** depths
    angle_rads = positions * angle_rates
    return np.concatenate(
        [np.sin(angle_rads), np.cos(angle_rads)], axis=-1
    ).astype(np.float32)


_POS = _positional_encoding(_SEQ, _DIM).reshape(-1)


@jax.jit
def _embed_sc(x1, table_p, pos):
    mesh = plsc.VectorSubcoreMesh(
        core_axis_name="core", subcore_axis_name="subcore"
    )

    @functools.partial(
        pl.kernel,
        out_type=jax.ShapeDtypeStruct((_BATCH * _SEQ, _DIM), jnp.float32),
        mesh=mesh,
        scratch_types=[
            pltpu.VMEM((_SEQ * _DIM,), jnp.float32),
            pltpu.VMEM((_SPW * _SEQ,), jnp.int32),
            [pltpu.VMEM((_G0, _PAD), jnp.float32) for _ in range(_RING)],
            [pltpu.VMEM((_G0, _DIM), jnp.float32) for _ in range(2)],
            [pltpu.SemaphoreType.DMA for _ in range(_RING)],
            [pltpu.SemaphoreType.DMA for _ in range(2)],
        ],
        compiler_params=pltpu.CompilerParams(use_tc_tiling_on_sc=True),
    )
    def k(x_hbm, table_hbm, pos_hbm, o_hbm,
          pos_v, idx_v, rows, obufs, gsems, osems):
        cid = lax.axis_index("core")
        sid = lax.axis_index("subcore")
        base = (sid * 2 + cid) * _SPW

        pltpu.sync_copy(pos_hbm, pos_v)
        pltpu.sync_copy(x_hbm.at[pl.ds(base * _SEQ, _SPW * _SEQ)], idx_v)

        def gather_desc(seq, half, b):
            n = _G0 if half == 0 else _G1
            i0 = seq * _SEQ + half * _G0
            return pltpu.make_async_copy(
                table_hbm.at[idx_v.at[pl.ds(i0, n)]],
                rows[b].at[pl.ds(0, n)],
                gsems[b],
            )

        def out_desc(seq, half):
            n = _G0 if half == 0 else _G1
            row0 = (base + seq) * _SEQ + half * _G0
            return pltpu.make_async_copy(
                obufs[half].at[pl.ds(0, n)],
                o_hbm.at[pl.ds(row0, n)],
                osems[half],
            )

        def compute(half, b):
            n = _G0 if half == 0 else _G1
            t0 = half * _G0
            src = rows[b]
            dst = obufs[half]

            @plsc.parallel_loop(0, n, step=2, unroll=4)
            def _(r):
                for rr in range(2):
                    row_s = src.at[r + rr]
                    row_d = dst.at[r + rr]
                    p0 = (t0 + r + rr) * _DIM
                    for c in range(0, _DIM, _LANES):
                        row_d.at[pl.ds(c, _LANES)][...] = (
                            row_s.at[pl.ds(c, _LANES)][...] * 8.0
                            + pos_v.at[pl.ds(p0 + c, _LANES)][...]
                        )

        gather_desc(0, 0, 0).start()
        gather_desc(0, 1, 1).start()

        @pl.loop(0, _ITEMS, step=_RING)
        def _(j):
            seq0 = j // 2
            for b in range(_RING):
                half = b % 2
                seq = seq0 + b // 2
                gather_desc(seq, half, b).wait()
                if b < 2:
                    @pl.when(j > 0)
                    def _():
                        out_desc(seq - 1, half).wait()
                else:
                    out_desc(seq - 1, half).wait()
                compute(half, b)
                out_desc(seq, half).start()

                q2 = j + b + 2

                @pl.when(q2 < _ITEMS)
                def _():
                    gather_desc(seq + 1, half, (b + 2) % _RING).start()

        out_desc(_SPW - 1, 0).wait()
        out_desc(_SPW - 1, 1).wait()

    return k(x1, table_p, pos)


def kernel(x, table):
    x1 = x.astype(jnp.int32).reshape(-1)
    table_p = jnp.pad(table, ((0, 0), (0, _PAD - _DIM)))
    out = _embed_sc(x1, table_p, jnp.asarray(_POS))
    return out.reshape(_BATCH, _SEQ, _DIM)


# speedup vs baseline: 1.9648x; 1.0118x over previous
import functools

import numpy as np
import jax
from jax import lax
import jax.numpy as jnp
from jax.experimental import pallas as pl
from jax.experimental.pallas import tpu as pltpu
from jax.experimental.pallas import tpu_sc as plsc

_DIM = 64
_PAD = 128
_SEQ = 200
_BATCH = 4096
_LANES = 16
_NW = 32
_SPW = _BATCH // _NW
_G0 = 104
_G1 = _SEQ - _G0
_ITEMS = _SPW * 2
_RING = 4


def _positional_encoding(length, dim):
    depth = dim / 2
    positions = np.arange(length)[:, np.newaxis]
    depths = np.arange(int(depth))[np.newaxis, :] / depth
    angle_rates = 1 / 10000 ** depths
    angle_rads = positions * angle_rates
    return np.concatenate(
        [np.sin(angle_rads), np.cos(angle_rads)], axis=-1
    ).astype(np.float32)


_POS = _positional_encoding(_SEQ, _DIM).reshape(-1)


@jax.jit
def _embed_sc(x1, table_p, pos):
    mesh = plsc.VectorSubcoreMesh(
        core_axis_name="core", subcore_axis_name="subcore"
    )

    @functools.partial(
        pl.kernel,
        out_type=jax.ShapeDtypeStruct((_BATCH * _SEQ, _DIM), jnp.float32),
        mesh=mesh,
        scratch_types=[
            pltpu.VMEM((_SEQ * _DIM,), jnp.float32),
            pltpu.VMEM((_SPW * _SEQ,), jnp.int32),
            [pltpu.VMEM((_G0, _PAD), jnp.float32) for _ in range(_RING)],
            [pltpu.VMEM((_G0, _DIM), jnp.float32) for _ in range(2)],
            [pltpu.SemaphoreType.DMA for _ in range(_RING)],
            [pltpu.SemaphoreType.DMA for _ in range(2)],
        ],
        compiler_params=pltpu.CompilerParams(use_tc_tiling_on_sc=True),
    )
    def k(x_hbm, table_hbm, pos_hbm, o_hbm,
          pos_v, idx_v, rows, obufs, gsems, osems):
        cid = lax.axis_index("core")
        sid = lax.axis_index("subcore")
        base = (sid * 2 + cid) * _SPW

        pltpu.sync_copy(pos_hbm, pos_v)
        pltpu.sync_copy(x_hbm.at[pl.ds(base * _SEQ, _SPW * _SEQ)], idx_v)

        def gather_desc(seq, half, b):
            n = _G0 if half == 0 else _G1
            i0 = seq * _SEQ + half * _G0
            return pltpu.make_async_copy(
                table_hbm.at[idx_v.at[pl.ds(i0, n)]],
                rows[b].at[pl.ds(0, n)],
                gsems[b],
            )

        def out_desc(seq, half):
            n = _G0 if half == 0 else _G1
            row0 = (base + seq) * _SEQ + half * _G0
            return pltpu.make_async_copy(
                obufs[half].at[pl.ds(0, n)],
                o_hbm.at[pl.ds(row0, n)],
                osems[half],
            )

        def compute(half, b):
            n = _G0 if half == 0 else _G1
            t0 = half * _G0
            src = rows[b]
            dst = obufs[half]

            @plsc.parallel_loop(0, n, step=2, unroll=4)
            def _(r):
                for rr in range(2):
                    row_s = src.at[r + rr]
                    row_d = dst.at[r + rr]
                    p0 = (t0 + r + rr) * _DIM
                    for c in range(0, _DIM, _LANES):
                        row_d.at[pl.ds(c, _LANES)][...] = (
                            row_s.at[pl.ds(c, _LANES)][...] * 8.0
                            + pos_v.at[pl.ds(p0 + c, _LANES)][...]
                        )

        gather_desc(0, 0, 0).start()
        gather_desc(0, 1, 1).start()
        gather_desc(1, 0, 2).start()

        @pl.loop(0, _ITEMS, step=_RING)
        def _(j):
            seq0 = j // 2
            for b in range(_RING):
                half = b % 2
                seq = seq0 + b // 2
                gather_desc(seq, half, b).wait()
                if b < 2:
                    @pl.when(j > 0)
                    def _():
                        out_desc(seq - 1, half).wait()
                else:
                    out_desc(seq - 1, half).wait()
                compute(half, b)
                out_desc(seq, half).start()

                q3 = j + b + 3

                @pl.when(q3 < _ITEMS)
                def _():
                    gather_desc(
                        j // 2 + (b + 3) // 2, (b + 3) % 2, (b + 3) % _RING
                    ).start()

        out_desc(_SPW - 1, 0).wait()
        out_desc(_SPW - 1, 1).wait()

    return k(x1, table_p, pos)


def kernel(x, table):
    x1 = x.astype(jnp.int32).reshape(-1)
    table_p = jnp.pad(table, ((0, 0), (0, _PAD - _DIM)))
    out = _embed_sc(x1, table_p, jnp.asarray(_POS))
    return out.reshape(_BATCH, _SEQ, _DIM)
